# baseline (speedup 1.0000x reference)
import jax
import jax.numpy as jnp
from jax.experimental import pallas as pl
from jax.experimental.pallas import tpu as pltpu

_IB = 2


def _fused_body(x_ref, o1_ref, o2_ref, o3_ref, o4_ref,
                a_s, b1_s, b2_s,
                w1_s, w2_s, w3_s, w4_s):
    @pl.when(pl.program_id(0) == 0)
    def _prep():
        x = x_ref[...]

        def cls(base):
            return x[:, base : base + 1, :] * x[:, base + 1 : base + 21, :]

        a_s[:, :, 0:20] = jnp.transpose(cls(0), (2, 0, 1))
        a_s[:, :, 20:40] = jnp.transpose(cls(51), (2, 0, 1))
        b1_s[...] = jnp.transpose(0.5 * cls(102), (2, 0, 1))
        b2_s[...] = jnp.transpose(0.5 * cls(153), (2, 0, 1))

        s_row = jax.lax.broadcasted_iota(jnp.int32, (14, 196), 0)
        p_col = jax.lax.broadcasted_iota(jnp.int32, (14, 196), 1)
        Rt = (p_col // 14 == s_row).astype(jnp.float32)
        Tt = (p_col % 14 == s_row).astype(jnp.float32)

        def sel(slab, onehot):
            return jax.lax.dot_general(
                slab, onehot, (((1,), (0,)), ((), ())),
                preferred_element_type=jnp.float32,
                precision=jax.lax.Precision.HIGHEST,
            )

        zy1 = jnp.transpose(x[:, 139:153, :], (1, 0, 2))
        zy2 = jnp.transpose(x[:, 190:204, :], (1, 0, 2))
        for c in range(14):
            lo, hi = 14 * c, 14 * (c + 1)
            cx1 = jnp.transpose(x[:, 23:37, lo:hi], (2, 0, 1)).reshape(224, 14)
            cy1 = jnp.transpose(x[:, 37:51, lo:hi], (2, 0, 1)).reshape(224, 14)
            cx2 = jnp.transpose(x[:, 74:88, lo:hi], (2, 0, 1)).reshape(224, 14)
            cy2 = jnp.transpose(x[:, 88:102, lo:hi], (2, 0, 1)).reshape(224, 14)
            lc1 = (sel(cx1, Rt) * sel(cy1, Tt)).reshape(14, 16, 196)
            lc2 = (sel(cx2, Rt) * sel(cy2, Tt)).reshape(14, 16, 196)
            lz1 = x[:, 125 + c, :][None, :, :] * zy1
            lz2 = x[:, 176 + c, :][None, :, :] * zy2
            w1_s[lo:hi] = lc1 * lz1
            w2_s[lo:hi] = lc2 * lz1
            w3_s[lo:hi] = lc1 * lz2
            w4_s[lo:hi] = lc2 * lz2

    B1 = b1_s[...]
    B2 = b2_s[...]
    i0 = pl.program_id(0) * _IB
    for k in range(_IB):
        i = i0 + k
        A1 = a_s[i, :, 0:20]
        A2 = a_s[i, :, 20:40]

        def emit(o_ref, w_s, A, B):
            WT = jnp.transpose(w_s[i])
            o_ref[k] = (WT[:, :, None] * A[None, :, :]) * B

        emit(o1_ref, w1_s, A1, B1)
        emit(o2_ref, w2_s, A2, B1)
        emit(o3_ref, w3_s, A1, B2)
        emit(o4_ref, w4_s, A2, B2)


def kernel(inference):
    N = inference.shape[0]
    inf = inference.reshape(N, 204, 196)
    f32 = jnp.float32
    scratch = (
        [pltpu.VMEM((196, 16, 40), f32)]
        + [pltpu.VMEM((196, 16, 20), f32)] * 2
        + [pltpu.VMEM((196, 16, 196), f32)] * 4
    )
    outs = pl.pallas_call(
        _fused_body,
        grid=(196 // _IB,),
        in_specs=[pl.BlockSpec((N, 204, 196), lambda i: (0, 0, 0))],
        out_specs=[pl.BlockSpec((_IB, 196, 16, 20), lambda i: (i, 0, 0, 0))] * 4,
        out_shape=[jax.ShapeDtypeStruct((196, 196, 16, 20), f32)] * 4,
        scratch_shapes=scratch,
        compiler_params=pltpu.CompilerParams(
            dimension_semantics=("arbitrary",),
        ),
    )(inf)
    return tuple(
        jnp.transpose(o, (2, 3, 0, 1)).reshape(N, 20, 14, 14, 14, 14) for o in outs
    )


# speedup vs baseline: 1.5520x; 1.5520x over previous
import jax
import jax.numpy as jnp
from jax.experimental import pallas as pl
from jax.experimental.pallas import tpu as pltpu


def _prep_body(x_ref, a1_ref, a2_ref, b1_ref, b2_ref,
               cx1_ref, cy1_ref, cx2_ref, cy2_ref,
               zx1_ref, zy1_ref, zx2_ref, zy2_ref):
    x = x_ref[...]

    def cls(base):
        return x[:, base : base + 1, :] * x[:, base + 1 : base + 21, :]

    a1_ref[...] = jnp.transpose(cls(0), (2, 0, 1))
    a2_ref[...] = jnp.transpose(cls(51), (2, 0, 1))
    cx1_ref[...] = jnp.transpose(x[:, 23:37, :], (2, 0, 1))
    cy1_ref[...] = jnp.transpose(x[:, 37:51, :], (2, 0, 1))
    cx2_ref[...] = jnp.transpose(x[:, 74:88, :], (2, 0, 1))
    cy2_ref[...] = jnp.transpose(x[:, 88:102, :], (2, 0, 1))
    b1_ref[...] = jnp.transpose(0.5 * cls(102), (2, 0, 1))
    b2_ref[...] = jnp.transpose(0.5 * cls(153), (2, 0, 1))
    zx1_ref[...] = jnp.transpose(x[:, 125:139, :], (1, 0, 2))
    zy1_ref[...] = jnp.transpose(x[:, 139:153, :], (1, 0, 2))
    zx2_ref[...] = jnp.transpose(x[:, 176:190, :], (1, 0, 2))
    zy2_ref[...] = jnp.transpose(x[:, 190:204, :], (1, 0, 2))


_IB = 4


def _big_body(a1_ref, a2_ref, b1_ref, b2_ref,
              cx1_ref, cy1_ref, cx2_ref, cy2_ref,
              zx1_ref, zy1_ref, zx2_ref, zy2_ref,
              o1_ref, o2_ref, o3_ref, o4_ref):
    s_row = jax.lax.broadcasted_iota(jnp.int32, (14, 196), 0)
    p_col = jax.lax.broadcasted_iota(jnp.int32, (14, 196), 1)
    Rt = (p_col // 14 == s_row).astype(jnp.float32)
    Tt = (p_col % 14 == s_row).astype(jnp.float32)

    def sel(slab, onehot):
        return jax.lax.dot_general(
            slab, onehot, (((1,), (0,)), ((), ())),
            preferred_element_type=jnp.float32,
            precision=jax.lax.Precision.HIGHEST,
        )

    Lc1 = sel(cx1_ref[...].reshape(16 * _IB, 14), Rt) * sel(
        cy1_ref[...].reshape(16 * _IB, 14), Tt)
    Lc2 = sel(cx2_ref[...].reshape(16 * _IB, 14), Rt) * sel(
        cy2_ref[...].reshape(16 * _IB, 14), Tt)

    B1 = b1_ref[...]
    B2 = b2_ref[...]

    i0 = pl.program_id(0) * _IB
    for k in range(_IB):
        i = i0 + k
        ix = jax.lax.div(i, 14)
        iy = jax.lax.rem(i, 14)
        Lz1 = zx1_ref[ix] * zy1_ref[iy]
        Lz2 = zx2_ref[ix] * zy2_ref[iy]
        lo, hi = 16 * k, 16 * (k + 1)
        Lc1k = Lc1[lo:hi]
        Lc2k = Lc2[lo:hi]
        A1 = a1_ref[k]
        A2 = a2_ref[k]

        def emit(o_ref, W, A, B):
            WT = jnp.transpose(W)
            o_ref[k] = (WT[:, :, None] * A[None, :, :]) * B

        emit(o1_ref, Lc1k * Lz1, A1, B1)
        emit(o2_ref, Lc2k * Lz1, A2, B1)
        emit(o3_ref, Lc1k * Lz2, A1, B2)
        emit(o4_ref, Lc2k * Lz2, A2, B2)


def kernel(inference):
    N = inference.shape[0]
    inf = inference.reshape(N, 204, 196)
    f32 = jnp.float32
    prep = pl.pallas_call(
        _prep_body,
        out_shape=[jax.ShapeDtypeStruct((196, 16, 20), f32)] * 4
        + [jax.ShapeDtypeStruct((196, 16, 14), f32)] * 4
        + [jax.ShapeDtypeStruct((14, 16, 196), f32)] * 4,
    )(inf)
    a1, a2, b1, b2, cx1, cy1, cx2, cy2, zx1, zy1, zx2, zy2 = prep

    slab20 = pl.BlockSpec((_IB, 16, 20), lambda i: (i, 0, 0))
    slab14 = pl.BlockSpec((_IB, 16, 14), lambda i: (i, 0, 0))
    full20 = pl.BlockSpec((196, 16, 20), lambda i: (0, 0, 0))
    fullz = pl.BlockSpec((14, 16, 196), lambda i: (0, 0, 0))
    outs = pl.pallas_call(
        _big_body,
        grid=(196 // _IB,),
        in_specs=[slab20, slab20, full20, full20,
                  slab14, slab14, slab14, slab14,
                  fullz, fullz, fullz, fullz],
        out_specs=[pl.BlockSpec((_IB, 196, 16, 20), lambda i: (i, 0, 0, 0))] * 4,
        out_shape=[jax.ShapeDtypeStruct((196, 196, 16, 20), f32)] * 4,
        compiler_params=pltpu.CompilerParams(
            dimension_semantics=("parallel",),
        ),
    )(a1, a2, b1, b2, cx1, cy1, cx2, cy2, zx1, zy1, zx2, zy2)
    return tuple(
        jnp.transpose(o, (2, 3, 0, 1)).reshape(N, 20, 14, 14, 14, 14) for o in outs
    )
